# baseline (speedup 1.0000x reference)
import functools

import jax
import jax.numpy as jnp
from jax import lax
from jax.experimental import pallas as pl
from jax.experimental.pallas import tpu as pltpu
from jax.experimental.pallas import tpu_sc as plsc

N = 10000
E = 320000
D = 128
G = 64
NLAYER = 5

NC = 2
NS = 16
NP = 10240
EP = 327680
CHUNK = 128
NCHUNK = EP // (NC * NS * CHUNK)
ROWS_PER_TILE = NP // NS
BLK = 256
NBLK = NP // BLK


SLAB = 8


def _sc_pipeline(vtab_hbm, vidx_hbm, didx_hbm, sid, acc_sh,
                 vidx_v, didx_v, bufs, gsems, ssems, nchunk):
    rows_a = bufs[0]

    @pl.loop(0, nchunk // SLAB)
    def _(o):
        pltpu.sync_copy(vidx_hbm.at[sid, pl.ds(o * SLAB, SLAB)], vidx_v)
        pltpu.sync_copy(didx_hbm.at[sid, pl.ds(o * SLAB, SLAB)], didx_v)
        gd = [None, None]
        sd = [None, None]
        gd[0] = pltpu.async_copy(vtab_hbm.at[vidx_v.at[0]], rows_a, gsems[0])
        for j in range(SLAB):
            b = j % 2
            if j + 1 < SLAB:
                if sd[1 - b] is not None:
                    sd[1 - b].wait()
                gd[1 - b] = pltpu.async_copy(vtab_hbm.at[vidx_v.at[j + 1]],
                                             bufs[1 - b], gsems[1 - b])
            gd[b].wait()
            sd[b] = pltpu.async_copy(bufs[b], acc_sh.at[didx_v.at[j]],
                                     ssems[b], add=True)
        sd[0].wait()
        sd[1].wait()


def _sc_scatter_body(nrows, width, k0, k1,
                     vtab_hbm, vidx0_hbm, didx0_hbm, vidx1_hbm, didx1_hbm,
                     out_hbm, vidx_v, didx_v, rows_a, rows_b,
                     sema, semb, semc, semd, acc_sh):
    cid = lax.axis_index("c")
    sid = lax.axis_index("s")
    rows_per_tile = nrows // NS

    @pl.loop(0, CHUNK)
    def _(r):
        for cc in range(width // 16):
            rows_a[r, pl.ds(cc * 16, 16)] = jnp.zeros((16,), jnp.float32)

    for k in range(rows_per_tile // CHUNK):
        pltpu.sync_copy(rows_a, acc_sh.at[pl.ds(sid * rows_per_tile + k * CHUNK, CHUNK)])
    plsc.subcore_barrier()

    bufs = (rows_a, rows_b)
    gsems = (sema, semb)
    ssems = (semc, semd)

    @pl.when(cid == 0)
    def _():
        _sc_pipeline(vtab_hbm, vidx0_hbm, didx0_hbm, sid, acc_sh,
                     vidx_v, didx_v, bufs, gsems, ssems, k0)

    @pl.when(cid == 1)
    def _():
        _sc_pipeline(vtab_hbm, vidx1_hbm, didx1_hbm, sid, acc_sh,
                     vidx_v, didx_v, bufs, gsems, ssems, k1)

    plsc.subcore_barrier()

    for k in range(rows_per_tile // CHUNK):
        base = sid * rows_per_tile + k * CHUNK
        pltpu.sync_copy(acc_sh.at[pl.ds(base, CHUNK)],
                        out_hbm.at[cid, pl.ds(base, CHUNK)])


def _sc_scatter(vtab, vidx0, didx0, vidx1, didx1, nrows, width):
    k0, k1 = vidx0.shape[1], vidx1.shape[1]
    mesh = plsc.VectorSubcoreMesh(core_axis_name="c", subcore_axis_name="s")
    kern = pl.kernel(
        functools.partial(_sc_scatter_body, nrows, width, k0, k1),
        out_type=jax.ShapeDtypeStruct((NC, nrows, width), jnp.float32),
        mesh=mesh,
        scratch_types=[
            pltpu.VMEM((SLAB, CHUNK), jnp.int32),
            pltpu.VMEM((SLAB, CHUNK), jnp.int32),
            pltpu.VMEM((CHUNK, width), jnp.float32),
            pltpu.VMEM((CHUNK, width), jnp.float32),
            pltpu.SemaphoreType.DMA,
            pltpu.SemaphoreType.DMA,
            pltpu.SemaphoreType.DMA,
            pltpu.SemaphoreType.DMA,
            pltpu.VMEM_SHARED((nrows, width), jnp.float32),
        ],
    )
    return kern(vtab, vidx0, didx0, vidx1, didx1)


def _embed_body(x0_ref, x1_ref, e1_ref, e2_ref, out_ref):
    b0 = x0_ref[...]
    b1 = x1_ref[...]
    acc = jnp.zeros((BLK, D), jnp.float32)
    iota = lax.broadcasted_iota(jnp.int32, (BLK, 128), 1)
    for k in range(4):
        m0 = (b0 == iota + 128 * k).astype(jnp.float32)
        m1 = (b1 == iota + 128 * k).astype(jnp.float32)
        acc += jnp.dot(m0, e1_ref[pl.ds(128 * k, 128), :],
                       preferred_element_type=jnp.float32, precision=lax.Precision.HIGHEST)
        acc += jnp.dot(m1, e2_ref[pl.ds(128 * k, 128), :],
                       preferred_element_type=jnp.float32, precision=lax.Precision.HIGHEST)
    out_ref[...] = acc


def _embed(x0b, x1b, e1p, e2p):
    return pl.pallas_call(
        _embed_body,
        grid=(NBLK,),
        in_specs=[
            pl.BlockSpec((BLK, 128), lambda i: (i, 0)),
            pl.BlockSpec((BLK, 128), lambda i: (i, 0)),
            pl.BlockSpec((512, D), lambda i: (0, 0)),
            pl.BlockSpec((512, D), lambda i: (0, 0)),
        ],
        out_specs=pl.BlockSpec((BLK, D), lambda i: (i, 0)),
        out_shape=jax.ShapeDtypeStruct((NP, D), jnp.float32),
    )(x0b, x1b, e1p, e2p)


def _mlp_body(a0_ref, a1_ref, h_ref, c0_ref, c1_ref, t_ref, r_ref,
              w1_ref, b1_ref, w2_ref, b2_ref, sg_ref, bt_ref, out_ref):
    cnt = c0_ref[...] + c1_ref[...]
    acc = (a0_ref[...] + a1_ref[...] + h_ref[...] + r_ref[...]
           + jnp.dot(cnt, t_ref[...], preferred_element_type=jnp.float32, precision=lax.Precision.HIGHEST))
    bf = jnp.bfloat16
    mid = jnp.maximum(
        jnp.dot(acc.astype(bf), w1_ref[...].astype(bf),
                preferred_element_type=jnp.float32)
        + b1_ref[...], 0.0)
    hn = (jnp.dot(mid.astype(bf), w2_ref[...].astype(bf),
                  preferred_element_type=jnp.float32)
          + b2_ref[...]) * sg_ref[...] + bt_ref[...]
    out_ref[...] = jnp.maximum(hn, 0.0)


def _mlp(a0, a1, h, c0, c1, t, r, w1, b1, w2, b2, sg, bt):
    full = lambda s: pl.BlockSpec(s, lambda i: tuple(0 for _ in s))
    return pl.pallas_call(
        _mlp_body,
        grid=(NBLK,),
        in_specs=[
            pl.BlockSpec((BLK, D), lambda i: (i, 0)),
            pl.BlockSpec((BLK, D), lambda i: (i, 0)),
            pl.BlockSpec((BLK, D), lambda i: (i, 0)),
            pl.BlockSpec((BLK, 128), lambda i: (i, 0)),
            pl.BlockSpec((BLK, 128), lambda i: (i, 0)),
            full((128, D)),
            full((1, D)),
            full((D, 2 * D)),
            full((1, 2 * D)),
            full((2 * D, D)),
            full((1, D)),
            full((1, D)),
            full((1, D)),
        ],
        out_specs=pl.BlockSpec((BLK, D), lambda i: (i, 0)),
        out_shape=jax.ShapeDtypeStruct((NP, D), jnp.float32),
    )(a0, a1, h, c0, c1, t, r, w1, b1, w2, b2, sg, bt)


def _pool_body(h_ref, bb_ref, w1_ref, b1_ref, w2_ref, b2_ref, out_ref,
               sums_ref, cnts_ref):
    i = pl.program_id(0)

    @pl.when(i == 0)
    def _():
        sums_ref[...] = jnp.zeros((128, D), jnp.float32)
        cnts_ref[...] = jnp.zeros((128, D), jnp.float32)

    bb = bb_ref[...]
    iota = lax.broadcasted_iota(jnp.int32, (BLK, 128), 1)
    onehot = (bb == iota).astype(jnp.float32)
    dn = (((0,), (0,)), ((), ()))
    sums_ref[...] += lax.dot_general(onehot, h_ref[...], dn,
                                     preferred_element_type=jnp.float32, precision=lax.Precision.HIGHEST)
    cnts_ref[...] += lax.dot_general(onehot, jnp.ones((BLK, D), jnp.float32), dn,
                                     preferred_element_type=jnp.float32, precision=lax.Precision.HIGHEST)

    @pl.when(i == NBLK - 1)
    def _():
        pooled = sums_ref[...] / jnp.maximum(cnts_ref[...], 1.0)
        bf = jnp.bfloat16
        z = (jnp.dot(pooled.astype(bf), w1_ref[...].astype(bf),
                     preferred_element_type=jnp.float32)
             + b1_ref[...])
        sp = jnp.maximum(z, 0.0) + jnp.log1p(jnp.exp(-jnp.abs(z)))
        res = (jnp.dot(sp.astype(bf), w2_ref[...].astype(bf),
                       preferred_element_type=jnp.float32)
               + b2_ref[...])
        out_ref[...] = res[:G, :]


def _pool(h, bb, w1, b1, w2, b2):
    full = lambda s: pl.BlockSpec(s, lambda i: tuple(0 for _ in s))
    return pl.pallas_call(
        _pool_body,
        grid=(NBLK,),
        in_specs=[
            pl.BlockSpec((BLK, D), lambda i: (i, 0)),
            pl.BlockSpec((BLK, 128), lambda i: (i, 0)),
            full((D, D // 2)),
            full((1, D // 2)),
            full((D // 2, 2)),
            full((1, 2)),
        ],
        out_specs=full((G, 2)),
        out_shape=jax.ShapeDtypeStruct((G, 2), jnp.float32),
        scratch_shapes=[
            pltpu.VMEM((128, D), jnp.float32),
            pltpu.VMEM((128, D), jnp.float32),
        ],
    )(h, bb, w1, b1, w2, b2)


def kernel(x, edge_index, edge_attr, batch, y, x_emb1, x_emb2, ee1s, ee2s,
           W1s, b1s, W2s, b2s, gammas, betas, out_W1, out_b1, out_W2, out_b2):
    f32 = jnp.float32

    src = edge_index[0].astype(jnp.int32)
    dst = edge_index[1].astype(jnp.int32)
    npad = EP - E
    pad_dst = (jnp.arange(npad, dtype=jnp.int32) % (NP - N)) + N
    srcp = jnp.concatenate([src, jnp.zeros((npad,), jnp.int32)])
    dstp = jnp.concatenate([dst, pad_dst])
    cls = (edge_attr[:, 0] * 3 + edge_attr[:, 1]).astype(jnp.int32)
    clsp = jnp.concatenate([cls, jnp.full((npad,), 15, jnp.int32)])
    clsp = clsp + 16 * (jnp.arange(EP, dtype=jnp.int32) % 64)
    K0, K1 = 32, 128
    e0 = NS * K0 * CHUNK

    def split(a):
        return (a[:e0].reshape(NS, K0, CHUNK), a[e0:].reshape(NS, K1, CHUNK))

    src0, src1 = split(srcp)
    dst0, dst1 = split(dstp)
    cls0, cls1 = split(clsp)

    x0b = jnp.broadcast_to(
        jnp.concatenate([x[:, 0].astype(jnp.int32), jnp.zeros((NP - N,), jnp.int32)])[:, None],
        (NP, 128))
    x1b = jnp.broadcast_to(
        jnp.concatenate([x[:, 1].astype(jnp.int32), jnp.zeros((NP - N,), jnp.int32)])[:, None],
        (NP, 128))
    batchb = jnp.broadcast_to(
        jnp.concatenate([batch.astype(jnp.int32), jnp.full((NP - N,), 10000, jnp.int32)])[:, None],
        (NP, 128))

    e1p = jnp.concatenate([x_emb1.astype(f32), jnp.zeros((12, D), f32)])
    e2p = jnp.concatenate([x_emb2.astype(f32), jnp.zeros((12, D), f32)])

    onehot128 = jnp.tile(jnp.concatenate([jnp.eye(15, 128, dtype=f32),
                                          jnp.zeros((1, 128), f32)]), (64, 1))

    i0 = jnp.repeat(jnp.arange(5), 3)[:15]
    i1 = jnp.tile(jnp.arange(3), 5)
    bn_scale = 1.0 / jnp.sqrt(1.0 + 1e-5)

    cnt = _sc_scatter(onehot128, cls0, dst0, cls1, dst1, NP, D)
    cnt0, cnt1 = cnt[0], cnt[1]

    h = _embed(x0b, x1b, e1p, e2p)

    h, cnt0, cnt1 = lax.optimization_barrier((h, cnt0, cnt1))

    for l in range(NLAYER):
        t_l = (ee1s[l][i0] + ee2s[l][i1]).astype(f32)
        t_l = jnp.concatenate([t_l, jnp.zeros((128 - 15, D), f32)])
        r_l = (ee1s[l][4] + ee2s[l][0]).astype(f32)[None, :]
        aggr = _sc_scatter(h, src0, dst0, src1, dst1, NP, D)
        h = _mlp(aggr[0], aggr[1], h, cnt0, cnt1, t_l, r_l,
                 W1s[l].astype(f32), b1s[l][None, :].astype(f32),
                 W2s[l].astype(f32), b2s[l][None, :].astype(f32),
                 (gammas[l] * bn_scale)[None, :].astype(f32),
                 betas[l][None, :].astype(f32))

    return _pool(h, batchb, out_W1.astype(f32), out_b1[None, :].astype(f32),
                 out_W2.astype(f32), out_b2[None, :].astype(f32))


# speedup vs baseline: 1.0452x; 1.0452x over previous
import functools

import jax
import jax.numpy as jnp
from jax import lax
from jax.experimental import pallas as pl
from jax.experimental.pallas import tpu as pltpu
from jax.experimental.pallas import tpu_sc as plsc

N = 10000
E = 320000
D = 128
G = 64
NLAYER = 5

NC = 2
NS = 16
NP = 10240
EP = 327680
CHUNK = 128
NCHUNK = EP // (NC * NS * CHUNK)
ROWS_PER_TILE = NP // NS
BLK = 256
NBLK = NP // BLK


SLAB = 8


def _sc_pipeline(vtab_hbm, vidx_hbm, didx_hbm, sid, acc_sh,
                 vidx_v, didx_v, bufs, gsems, ssems, nchunk):
    rows_a = bufs[0]

    @pl.loop(0, nchunk // SLAB)
    def _(o):
        pltpu.sync_copy(vidx_hbm.at[sid, pl.ds(o * SLAB, SLAB)], vidx_v)
        pltpu.sync_copy(didx_hbm.at[sid, pl.ds(o * SLAB, SLAB)], didx_v)
        gd = [None, None]
        sd = [None, None]
        gd[0] = pltpu.async_copy(vtab_hbm.at[vidx_v.at[0]], rows_a, gsems[0])
        for j in range(SLAB):
            b = j % 2
            if j + 1 < SLAB:
                if sd[1 - b] is not None:
                    sd[1 - b].wait()
                gd[1 - b] = pltpu.async_copy(vtab_hbm.at[vidx_v.at[j + 1]],
                                             bufs[1 - b], gsems[1 - b])
            gd[b].wait()
            sd[b] = pltpu.async_copy(bufs[b], acc_sh.at[didx_v.at[j]],
                                     ssems[b], add=True)
        sd[0].wait()
        sd[1].wait()


def _sc_scatter_body(nrows, width, k0, k1,
                     vtab_hbm, vidx0_hbm, didx0_hbm, vidx1_hbm, didx1_hbm,
                     out_hbm, vidx_v, didx_v, rows_a, rows_b,
                     sema, semb, semc, semd, acc_sh):
    cid = lax.axis_index("c")
    sid = lax.axis_index("s")
    rows_per_tile = nrows // NS

    @pl.loop(0, CHUNK)
    def _(r):
        for cc in range(width // 16):
            rows_a[r, pl.ds(cc * 16, 16)] = jnp.zeros((16,), jnp.float32)

    for k in range(rows_per_tile // CHUNK):
        pltpu.sync_copy(rows_a, acc_sh.at[pl.ds(sid * rows_per_tile + k * CHUNK, CHUNK)])
    plsc.subcore_barrier()

    bufs = (rows_a, rows_b)
    gsems = (sema, semb)
    ssems = (semc, semd)

    @pl.when(cid == 0)
    def _():
        _sc_pipeline(vtab_hbm, vidx0_hbm, didx0_hbm, sid, acc_sh,
                     vidx_v, didx_v, bufs, gsems, ssems, k0)

    @pl.when(cid == 1)
    def _():
        _sc_pipeline(vtab_hbm, vidx1_hbm, didx1_hbm, sid, acc_sh,
                     vidx_v, didx_v, bufs, gsems, ssems, k1)

    plsc.subcore_barrier()

    for k in range(rows_per_tile // CHUNK):
        base = sid * rows_per_tile + k * CHUNK
        pltpu.sync_copy(acc_sh.at[pl.ds(base, CHUNK)],
                        out_hbm.at[cid, pl.ds(base, CHUNK)])


def _sc_scatter(vtab, vidx0, didx0, vidx1, didx1, nrows, width):
    k0, k1 = vidx0.shape[1], vidx1.shape[1]
    mesh = plsc.VectorSubcoreMesh(core_axis_name="c", subcore_axis_name="s")
    kern = pl.kernel(
        functools.partial(_sc_scatter_body, nrows, width, k0, k1),
        out_type=jax.ShapeDtypeStruct((NC, nrows, width), jnp.float32),
        mesh=mesh,
        scratch_types=[
            pltpu.VMEM((SLAB, CHUNK), jnp.int32),
            pltpu.VMEM((SLAB, CHUNK), jnp.int32),
            pltpu.VMEM((CHUNK, width), jnp.float32),
            pltpu.VMEM((CHUNK, width), jnp.float32),
            pltpu.SemaphoreType.DMA,
            pltpu.SemaphoreType.DMA,
            pltpu.SemaphoreType.DMA,
            pltpu.SemaphoreType.DMA,
            pltpu.VMEM_SHARED((nrows, width), jnp.float32),
        ],
    )
    return kern(vtab, vidx0, didx0, vidx1, didx1)


def _embed_body(x0_ref, x1_ref, e1_ref, e2_ref, out_ref):
    b0 = x0_ref[...]
    b1 = x1_ref[...]
    acc = jnp.zeros((BLK, D), jnp.float32)
    iota = lax.broadcasted_iota(jnp.int32, (BLK, 128), 1)
    for k in range(4):
        m0 = (b0 == iota + 128 * k).astype(jnp.float32)
        m1 = (b1 == iota + 128 * k).astype(jnp.float32)
        acc += jnp.dot(m0, e1_ref[pl.ds(128 * k, 128), :],
                       preferred_element_type=jnp.float32, precision=lax.Precision.HIGHEST)
        acc += jnp.dot(m1, e2_ref[pl.ds(128 * k, 128), :],
                       preferred_element_type=jnp.float32, precision=lax.Precision.HIGHEST)
    out_ref[...] = acc


def _embed(x0b, x1b, e1p, e2p):
    return pl.pallas_call(
        _embed_body,
        grid=(NBLK,),
        in_specs=[
            pl.BlockSpec((BLK, 128), lambda i: (i, 0)),
            pl.BlockSpec((BLK, 128), lambda i: (i, 0)),
            pl.BlockSpec((512, D), lambda i: (0, 0)),
            pl.BlockSpec((512, D), lambda i: (0, 0)),
        ],
        out_specs=pl.BlockSpec((BLK, D), lambda i: (i, 0)),
        out_shape=jax.ShapeDtypeStruct((NP, D), jnp.float32),
    )(x0b, x1b, e1p, e2p)


def _mlp_body(a0_ref, a1_ref, h_ref, c0_ref, c1_ref, t_ref, r_ref,
              w1_ref, b1_ref, w2_ref, b2_ref, sg_ref, bt_ref, out_ref):
    cnt = c0_ref[...] + c1_ref[...]
    acc = (a0_ref[...] + a1_ref[...] + h_ref[...] + r_ref[...]
           + jnp.dot(cnt, t_ref[...], preferred_element_type=jnp.float32, precision=lax.Precision.HIGHEST))
    bf = jnp.bfloat16
    mid = jnp.maximum(
        jnp.dot(acc.astype(bf), w1_ref[...].astype(bf),
                preferred_element_type=jnp.float32)
        + b1_ref[...], 0.0)
    hn = (jnp.dot(mid.astype(bf), w2_ref[...].astype(bf),
                  preferred_element_type=jnp.float32)
          + b2_ref[...]) * sg_ref[...] + bt_ref[...]
    out_ref[...] = jnp.maximum(hn, 0.0)


def _mlp(a0, a1, h, c0, c1, t, r, w1, b1, w2, b2, sg, bt):
    full = lambda s: pl.BlockSpec(s, lambda i: tuple(0 for _ in s))
    return pl.pallas_call(
        _mlp_body,
        grid=(NBLK,),
        in_specs=[
            pl.BlockSpec((BLK, D), lambda i: (i, 0)),
            pl.BlockSpec((BLK, D), lambda i: (i, 0)),
            pl.BlockSpec((BLK, D), lambda i: (i, 0)),
            pl.BlockSpec((BLK, 128), lambda i: (i, 0)),
            pl.BlockSpec((BLK, 128), lambda i: (i, 0)),
            full((128, D)),
            full((1, D)),
            full((D, 2 * D)),
            full((1, 2 * D)),
            full((2 * D, D)),
            full((1, D)),
            full((1, D)),
            full((1, D)),
        ],
        out_specs=pl.BlockSpec((BLK, D), lambda i: (i, 0)),
        out_shape=jax.ShapeDtypeStruct((NP, D), jnp.float32),
    )(a0, a1, h, c0, c1, t, r, w1, b1, w2, b2, sg, bt)


def _pool_body(h_ref, bb_ref, w1_ref, b1_ref, w2_ref, b2_ref, out_ref,
               sums_ref, cnts_ref):
    i = pl.program_id(0)

    @pl.when(i == 0)
    def _():
        sums_ref[...] = jnp.zeros((128, D), jnp.float32)
        cnts_ref[...] = jnp.zeros((128, D), jnp.float32)

    bb = bb_ref[...]
    iota = lax.broadcasted_iota(jnp.int32, (BLK, 128), 1)
    onehot = (bb == iota).astype(jnp.float32)
    dn = (((0,), (0,)), ((), ()))
    sums_ref[...] += lax.dot_general(onehot, h_ref[...], dn,
                                     preferred_element_type=jnp.float32, precision=lax.Precision.HIGHEST)
    cnts_ref[...] += lax.dot_general(onehot, jnp.ones((BLK, D), jnp.float32), dn,
                                     preferred_element_type=jnp.float32, precision=lax.Precision.HIGHEST)

    @pl.when(i == NBLK - 1)
    def _():
        pooled = sums_ref[...] / jnp.maximum(cnts_ref[...], 1.0)
        bf = jnp.bfloat16
        z = (jnp.dot(pooled.astype(bf), w1_ref[...].astype(bf),
                     preferred_element_type=jnp.float32)
             + b1_ref[...])
        sp = jnp.maximum(z, 0.0) + jnp.log1p(jnp.exp(-jnp.abs(z)))
        res = (jnp.dot(sp.astype(bf), w2_ref[...].astype(bf),
                       preferred_element_type=jnp.float32)
               + b2_ref[...])
        out_ref[...] = res[:G, :]


def _pool(h, bb, w1, b1, w2, b2):
    full = lambda s: pl.BlockSpec(s, lambda i: tuple(0 for _ in s))
    return pl.pallas_call(
        _pool_body,
        grid=(NBLK,),
        in_specs=[
            pl.BlockSpec((BLK, D), lambda i: (i, 0)),
            pl.BlockSpec((BLK, 128), lambda i: (i, 0)),
            full((D, D // 2)),
            full((1, D // 2)),
            full((D // 2, 2)),
            full((1, 2)),
        ],
        out_specs=full((G, 2)),
        out_shape=jax.ShapeDtypeStruct((G, 2), jnp.float32),
        scratch_shapes=[
            pltpu.VMEM((128, D), jnp.float32),
            pltpu.VMEM((128, D), jnp.float32),
        ],
    )(h, bb, w1, b1, w2, b2)


def kernel(x, edge_index, edge_attr, batch, y, x_emb1, x_emb2, ee1s, ee2s,
           W1s, b1s, W2s, b2s, gammas, betas, out_W1, out_b1, out_W2, out_b2):
    f32 = jnp.float32

    src = edge_index[0].astype(jnp.int32)
    dst = edge_index[1].astype(jnp.int32)
    npad = EP - E
    pad_dst = (jnp.arange(npad, dtype=jnp.int32) % (NP - N)) + N
    srcp = jnp.concatenate([src, jnp.zeros((npad,), jnp.int32)])
    dstp = jnp.concatenate([dst, pad_dst])
    cls = (edge_attr[:, 0] * 3 + edge_attr[:, 1]).astype(jnp.int32)
    clsp = jnp.concatenate([cls, jnp.full((npad,), 15, jnp.int32)])
    clsp = clsp + 16 * (jnp.arange(EP, dtype=jnp.int32) % 64)
    K0, K1 = 80, 80
    e0 = NS * K0 * CHUNK

    def split(a):
        return (a[:e0].reshape(NS, K0, CHUNK), a[e0:].reshape(NS, K1, CHUNK))

    src0, src1 = split(srcp)
    dst0, dst1 = split(dstp)
    cls0, cls1 = split(clsp)

    x0b = jnp.broadcast_to(
        jnp.concatenate([x[:, 0].astype(jnp.int32), jnp.zeros((NP - N,), jnp.int32)])[:, None],
        (NP, 128))
    x1b = jnp.broadcast_to(
        jnp.concatenate([x[:, 1].astype(jnp.int32), jnp.zeros((NP - N,), jnp.int32)])[:, None],
        (NP, 128))
    batchb = jnp.broadcast_to(
        jnp.concatenate([batch.astype(jnp.int32), jnp.full((NP - N,), 10000, jnp.int32)])[:, None],
        (NP, 128))

    e1p = jnp.concatenate([x_emb1.astype(f32), jnp.zeros((12, D), f32)])
    e2p = jnp.concatenate([x_emb2.astype(f32), jnp.zeros((12, D), f32)])

    onehot128 = jnp.tile(jnp.concatenate([jnp.eye(15, 128, dtype=f32),
                                          jnp.zeros((1, 128), f32)]), (64, 1))

    i0 = jnp.repeat(jnp.arange(5), 3)[:15]
    i1 = jnp.tile(jnp.arange(3), 5)
    bn_scale = 1.0 / jnp.sqrt(1.0 + 1e-5)

    cnt = _sc_scatter(onehot128, cls0, dst0, cls1, dst1, NP, D)
    cnt0, cnt1 = cnt[0], cnt[1]

    h = _embed(x0b, x1b, e1p, e2p)

    h, cnt0, cnt1 = lax.optimization_barrier((h, cnt0, cnt1))

    for l in range(NLAYER):
        t_l = (ee1s[l][i0] + ee2s[l][i1]).astype(f32)
        t_l = jnp.concatenate([t_l, jnp.zeros((128 - 15, D), f32)])
        r_l = (ee1s[l][4] + ee2s[l][0]).astype(f32)[None, :]
        aggr = _sc_scatter(h, src0, dst0, src1, dst1, NP, D)
        h = _mlp(aggr[0], aggr[1], h, cnt0, cnt1, t_l, r_l,
                 W1s[l].astype(f32), b1s[l][None, :].astype(f32),
                 W2s[l].astype(f32), b2s[l][None, :].astype(f32),
                 (gammas[l] * bn_scale)[None, :].astype(f32),
                 betas[l][None, :].astype(f32))

    return _pool(h, batchb, out_W1.astype(f32), out_b1[None, :].astype(f32),
                 out_W2.astype(f32), out_b2[None, :].astype(f32))


# speedup vs baseline: 1.1848x; 1.1336x over previous
import functools

import jax
import jax.numpy as jnp
from jax import lax
from jax.experimental import pallas as pl
from jax.experimental.pallas import tpu as pltpu
from jax.experimental.pallas import tpu_sc as plsc

N = 10000
E = 320000
D = 128
G = 64
NLAYER = 5

NC = 2
NS = 16
NP = 10240
EP = 327680
CHUNK = 128
NCHUNK = EP // (NC * NS * CHUNK)
ROWS_PER_TILE = NP // NS
BLK = 256
NBLK = NP // BLK


SLAB = 8


def _sc_pipeline(vtab_hbm, vidx_hbm, didx_hbm, sid, acc_sh,
                 vidx_v, didx_v, bufs, gsems, ssems, nchunk):
    rows_a = bufs[0]

    @pl.loop(0, nchunk // SLAB)
    def _(o):
        pltpu.sync_copy(vidx_hbm.at[sid, pl.ds(o * SLAB, SLAB)], vidx_v)
        pltpu.sync_copy(didx_hbm.at[sid, pl.ds(o * SLAB, SLAB)], didx_v)
        gd = [None, None]
        sd = [None, None]
        gd[0] = pltpu.async_copy(vtab_hbm.at[vidx_v.at[0]], rows_a, gsems[0])
        for j in range(SLAB):
            b = j % 2
            if j + 1 < SLAB:
                if sd[1 - b] is not None:
                    sd[1 - b].wait()
                gd[1 - b] = pltpu.async_copy(vtab_hbm.at[vidx_v.at[j + 1]],
                                             bufs[1 - b], gsems[1 - b])
            gd[b].wait()
            sd[b] = pltpu.async_copy(bufs[b], acc_sh.at[didx_v.at[j]],
                                     ssems[b], add=True)
        sd[0].wait()
        sd[1].wait()


def _sc_scatter_body(nrows, width, vtab_hbm, vidx_hbm, didx_hbm, out_hbm,
                     vidx_v, didx_v, rows_a, rows_b, sema, semb, semc, semd,
                     acc_sh):
    cid = lax.axis_index("c")
    sid = lax.axis_index("s")
    rows_per_tile = nrows // NS

    @pl.loop(0, CHUNK)
    def _(r):
        for cc in range(width // 16):
            rows_a[r, pl.ds(cc * 16, 16)] = jnp.zeros((16,), jnp.float32)

    for k in range(rows_per_tile // CHUNK):
        pltpu.sync_copy(rows_a, acc_sh.at[pl.ds(sid * rows_per_tile + k * CHUNK, CHUNK)])
    plsc.subcore_barrier()

    bufs = (rows_a, rows_b)
    gsems = (sema, semb)
    ssems = (semc, semd)
    _sc_pipeline(vtab_hbm, vidx_hbm.at[cid], didx_hbm.at[cid], sid, acc_sh,
                 vidx_v, didx_v, bufs, gsems, ssems, NCHUNK)

    plsc.subcore_barrier()

    for k in range(rows_per_tile // CHUNK):
        base = sid * rows_per_tile + k * CHUNK
        pltpu.sync_copy(acc_sh.at[pl.ds(base, CHUNK)],
                        out_hbm.at[cid, pl.ds(base, CHUNK)])


def _sc_scatter(vtab, vidx, didx, nrows, width):
    mesh = plsc.VectorSubcoreMesh(core_axis_name="c", subcore_axis_name="s")
    kern = pl.kernel(
        functools.partial(_sc_scatter_body, nrows, width),
        out_type=jax.ShapeDtypeStruct((NC, nrows, width), jnp.float32),
        mesh=mesh,
        scratch_types=[
            pltpu.VMEM((SLAB, CHUNK), jnp.int32),
            pltpu.VMEM((SLAB, CHUNK), jnp.int32),
            pltpu.VMEM((CHUNK, width), jnp.float32),
            pltpu.VMEM((CHUNK, width), jnp.float32),
            pltpu.SemaphoreType.DMA,
            pltpu.SemaphoreType.DMA,
            pltpu.SemaphoreType.DMA,
            pltpu.SemaphoreType.DMA,
            pltpu.VMEM_SHARED((nrows, width), jnp.float32),
        ],
    )
    return kern(vtab, vidx, didx)


def _embed_body(x0_ref, x1_ref, e1_ref, e2_ref, out_ref):
    b0 = x0_ref[...]
    b1 = x1_ref[...]
    acc = jnp.zeros((BLK, D), jnp.float32)
    iota = lax.broadcasted_iota(jnp.int32, (BLK, 128), 1)
    for k in range(4):
        m0 = (b0 == iota + 128 * k).astype(jnp.float32)
        m1 = (b1 == iota + 128 * k).astype(jnp.float32)
        acc += jnp.dot(m0, e1_ref[pl.ds(128 * k, 128), :],
                       preferred_element_type=jnp.float32, precision=lax.Precision.HIGHEST)
        acc += jnp.dot(m1, e2_ref[pl.ds(128 * k, 128), :],
                       preferred_element_type=jnp.float32, precision=lax.Precision.HIGHEST)
    out_ref[...] = acc


def _embed(x0b, x1b, e1p, e2p):
    return pl.pallas_call(
        _embed_body,
        grid=(NBLK,),
        in_specs=[
            pl.BlockSpec((BLK, 128), lambda i: (i, 0)),
            pl.BlockSpec((BLK, 128), lambda i: (i, 0)),
            pl.BlockSpec((512, D), lambda i: (0, 0)),
            pl.BlockSpec((512, D), lambda i: (0, 0)),
        ],
        out_specs=pl.BlockSpec((BLK, D), lambda i: (i, 0)),
        out_shape=jax.ShapeDtypeStruct((NP, D), jnp.float32),
    )(x0b, x1b, e1p, e2p)


def _mlp_body(a0_ref, a1_ref, h_ref, c0_ref, c1_ref, t_ref, r_ref,
              w1_ref, b1_ref, w2_ref, b2_ref, sg_ref, bt_ref, out_ref):
    cnt = c0_ref[...] + c1_ref[...]
    acc = (a0_ref[...] + a1_ref[...] + h_ref[...] + r_ref[...]
           + jnp.dot(cnt, t_ref[...], preferred_element_type=jnp.float32, precision=lax.Precision.HIGHEST))
    bf = jnp.bfloat16
    mid = jnp.maximum(
        jnp.dot(acc.astype(bf), w1_ref[...].astype(bf),
                preferred_element_type=jnp.float32)
        + b1_ref[...], 0.0)
    hn = (jnp.dot(mid.astype(bf), w2_ref[...].astype(bf),
                  preferred_element_type=jnp.float32)
          + b2_ref[...]) * sg_ref[...] + bt_ref[...]
    out_ref[...] = jnp.maximum(hn, 0.0)


def _mlp(a0, a1, h, c0, c1, t, r, w1, b1, w2, b2, sg, bt):
    full = lambda s: pl.BlockSpec(s, lambda i: tuple(0 for _ in s))
    return pl.pallas_call(
        _mlp_body,
        grid=(NBLK,),
        in_specs=[
            pl.BlockSpec((BLK, D), lambda i: (i, 0)),
            pl.BlockSpec((BLK, D), lambda i: (i, 0)),
            pl.BlockSpec((BLK, D), lambda i: (i, 0)),
            pl.BlockSpec((BLK, 128), lambda i: (i, 0)),
            pl.BlockSpec((BLK, 128), lambda i: (i, 0)),
            full((128, D)),
            full((1, D)),
            full((D, 2 * D)),
            full((1, 2 * D)),
            full((2 * D, D)),
            full((1, D)),
            full((1, D)),
            full((1, D)),
        ],
        out_specs=pl.BlockSpec((BLK, D), lambda i: (i, 0)),
        out_shape=jax.ShapeDtypeStruct((NP, D), jnp.float32),
    )(a0, a1, h, c0, c1, t, r, w1, b1, w2, b2, sg, bt)


def _pool_body(h_ref, bb_ref, w1_ref, b1_ref, w2_ref, b2_ref, out_ref,
               sums_ref, cnts_ref):
    i = pl.program_id(0)

    @pl.when(i == 0)
    def _():
        sums_ref[...] = jnp.zeros((128, D), jnp.float32)
        cnts_ref[...] = jnp.zeros((128, D), jnp.float32)

    bb = bb_ref[...]
    iota = lax.broadcasted_iota(jnp.int32, (BLK, 128), 1)
    onehot = (bb == iota).astype(jnp.float32)
    dn = (((0,), (0,)), ((), ()))
    sums_ref[...] += lax.dot_general(onehot, h_ref[...], dn,
                                     preferred_element_type=jnp.float32, precision=lax.Precision.HIGHEST)
    cnts_ref[...] += lax.dot_general(onehot, jnp.ones((BLK, D), jnp.float32), dn,
                                     preferred_element_type=jnp.float32, precision=lax.Precision.HIGHEST)

    @pl.when(i == NBLK - 1)
    def _():
        pooled = sums_ref[...] / jnp.maximum(cnts_ref[...], 1.0)
        bf = jnp.bfloat16
        z = (jnp.dot(pooled.astype(bf), w1_ref[...].astype(bf),
                     preferred_element_type=jnp.float32)
             + b1_ref[...])
        sp = jnp.maximum(z, 0.0) + jnp.log1p(jnp.exp(-jnp.abs(z)))
        res = (jnp.dot(sp.astype(bf), w2_ref[...].astype(bf),
                       preferred_element_type=jnp.float32)
               + b2_ref[...])
        out_ref[...] = res[:G, :]


def _pool(h, bb, w1, b1, w2, b2):
    full = lambda s: pl.BlockSpec(s, lambda i: tuple(0 for _ in s))
    return pl.pallas_call(
        _pool_body,
        grid=(NBLK,),
        in_specs=[
            pl.BlockSpec((BLK, D), lambda i: (i, 0)),
            pl.BlockSpec((BLK, 128), lambda i: (i, 0)),
            full((D, D // 2)),
            full((1, D // 2)),
            full((D // 2, 2)),
            full((1, 2)),
        ],
        out_specs=full((G, 2)),
        out_shape=jax.ShapeDtypeStruct((G, 2), jnp.float32),
        scratch_shapes=[
            pltpu.VMEM((128, D), jnp.float32),
            pltpu.VMEM((128, D), jnp.float32),
        ],
    )(h, bb, w1, b1, w2, b2)


def kernel(x, edge_index, edge_attr, batch, y, x_emb1, x_emb2, ee1s, ee2s,
           W1s, b1s, W2s, b2s, gammas, betas, out_W1, out_b1, out_W2, out_b2):
    f32 = jnp.float32

    src = edge_index[0].astype(jnp.int32)
    dst = edge_index[1].astype(jnp.int32)
    npad = EP - E
    pad_dst = (jnp.arange(npad, dtype=jnp.int32) % (NP - N)) + N
    srcp = jnp.concatenate([src, jnp.zeros((npad,), jnp.int32)])
    dstp = jnp.concatenate([dst, pad_dst])
    cls = (edge_attr[:, 0] * 3 + edge_attr[:, 1]).astype(jnp.int32)
    clsp = jnp.concatenate([cls, jnp.full((npad,), 15, jnp.int32)])
    clsp = clsp + 16 * (jnp.arange(EP, dtype=jnp.int32) % 64)
    src4 = srcp.reshape(NC, NS, NCHUNK, CHUNK)
    dst4 = dstp.reshape(NC, NS, NCHUNK, CHUNK)
    cls4 = clsp.reshape(NC, NS, NCHUNK, CHUNK)

    x0b = jnp.broadcast_to(
        jnp.concatenate([x[:, 0].astype(jnp.int32), jnp.zeros((NP - N,), jnp.int32)])[:, None],
        (NP, 128))
    x1b = jnp.broadcast_to(
        jnp.concatenate([x[:, 1].astype(jnp.int32), jnp.zeros((NP - N,), jnp.int32)])[:, None],
        (NP, 128))
    batchb = jnp.broadcast_to(
        jnp.concatenate([batch.astype(jnp.int32), jnp.full((NP - N,), 10000, jnp.int32)])[:, None],
        (NP, 128))

    e1p = jnp.concatenate([x_emb1.astype(f32), jnp.zeros((12, D), f32)])
    e2p = jnp.concatenate([x_emb2.astype(f32), jnp.zeros((12, D), f32)])

    onehot128 = jnp.tile(jnp.concatenate([jnp.eye(15, 128, dtype=f32),
                                          jnp.zeros((1, 128), f32)]), (64, 1))

    i0 = jnp.repeat(jnp.arange(5), 3)[:15]
    i1 = jnp.tile(jnp.arange(3), 5)
    bn_scale = 1.0 / jnp.sqrt(1.0 + 1e-5)

    cnt = _sc_scatter(onehot128, cls4, dst4, NP, D)
    cnt0, cnt1 = cnt[0], cnt[1]

    h = _embed(x0b, x1b, e1p, e2p)

    h, cnt0, cnt1 = lax.optimization_barrier((h, cnt0, cnt1))

    for l in range(NLAYER):
        t_l = (ee1s[l][i0] + ee2s[l][i1]).astype(f32)
        t_l = jnp.concatenate([t_l, jnp.zeros((128 - 15, D), f32)])
        r_l = (ee1s[l][4] + ee2s[l][0]).astype(f32)[None, :]
        aggr = _sc_scatter(h, src4, dst4, NP, D)
        h = _mlp(aggr[0], aggr[1], h, cnt0, cnt1, t_l, r_l,
                 W1s[l].astype(f32), b1s[l][None, :].astype(f32),
                 W2s[l].astype(f32), b2s[l][None, :].astype(f32),
                 (gammas[l] * bn_scale)[None, :].astype(f32),
                 betas[l][None, :].astype(f32))

    return _pool(h, batchb, out_W1.astype(f32), out_b1[None, :].astype(f32),
                 out_W2.astype(f32), out_b2[None, :].astype(f32))
